# baseline (speedup 1.0000x reference)
import jax
import jax.numpy as jnp
from jax import lax
from jax.experimental import pallas as pl
from jax.experimental.pallas import tpu as pltpu

_N = 5000
_T = 512
_NBLK = 10
_NPAD = _T * _NBLK
_THR = 0.7


def _nms_body(bt_ref, bc_ref, out_ref, keep_ref):
    upper = (
        lax.broadcasted_iota(jnp.int32, (_T, _T), 0)
        < lax.broadcasted_iota(jnp.int32, (_T, _T), 1)
    ).astype(jnp.float32)
    colc = lax.broadcasted_iota(jnp.int32, (_NPAD, 1), 0).astype(jnp.float32)
    keep_ref[...] = jnp.ones((8, _NPAD), jnp.float32)
    out_ref[...] = jnp.zeros((_NPAD, 8), jnp.float32)

    _C = _THR / (1.0 + _THR)
    x1f = bt_ref[0:1, :]
    y1f = bt_ref[1:2, :]
    x2f = bt_ref[2:3, :]
    y2f = bt_ref[3:4, :]
    careaf = _C * ((x2f - x1f) * (y2f - y1f))

    for j in range(_NBLK):
        b = j * _T
        x1 = x1f[:, b:]
        y1 = y1f[:, b:]
        x2 = x2f[:, b:]
        y2 = y2f[:, b:]
        carea = careaf[:, b:]
        rx1 = bc_ref[b : b + _T, 0:1]
        ry1 = bc_ref[b : b + _T, 1:2]
        rx2 = bc_ref[b : b + _T, 2:3]
        ry2 = bc_ref[b : b + _T, 3:4]
        crarea = _C * ((rx2 - rx1) * (ry2 - ry1))
        xl = jnp.minimum(rx2, x2) - jnp.maximum(rx1, x1)
        yl = jnp.minimum(ry2, y2) - jnp.maximum(ry1, y1)
        inter = xl * jnp.maximum(yl, 0.0)
        smat = (inter > crarea + carea).astype(jnp.float32)

        diag = smat[:, 0:_T] * upper
        kb0 = keep_ref[0:1, b : b + _T]

        def cond(c):
            return c[2]

        def body(c):
            kb, _, _ = c
            s = lax.dot(kb, diag, preferred_element_type=jnp.float32)
            kbn = jnp.where(s > 0.0, 0.0, kb0)
            return (kbn, kb, jnp.any(kbn != kb))

        kb = lax.while_loop(cond, body, (kb0, kb0, jnp.bool_(True)))[0]

        keep_ref[0:1, b : b + _T] = kb
        if j < _NBLK - 1:
            sup = lax.dot(kb, smat, preferred_element_type=jnp.float32)
            lcol = lax.broadcasted_iota(jnp.int32, (1, _NPAD - b), 1)
            keep = keep_ref[0:1, b:]
            keep_ref[0:1, b:] = jnp.where(
                (lcol >= _T) & (sup > 0.0), 0.0, keep
            )

        ordl = bt_ref[4:5, b : b + _T]
        ptile_t = (colc == ordl).astype(jnp.float32)
        kbt = jnp.reshape(kb, (_T, 1))
        rsc = bc_ref[b : b + _T, 5:6]
        lhs = jnp.concatenate(
            [kbt * rx1, kbt * ry1, kbt * rx2, kbt * ry2, kbt * rsc],
            axis=1,
        )
        out_ref[:, 0:5] += lax.dot(
            ptile_t,
            lhs,
            preferred_element_type=jnp.float32,
            precision=lax.Precision.HIGHEST,
        )


def kernel(boxes, scores):
    iota = jnp.arange(_N, dtype=jnp.float32)
    key, ordf, sx1, sy1, sx2, sy2 = lax.sort(
        (-scores, iota, boxes[:, 0], boxes[:, 1], boxes[:, 2], boxes[:, 3]),
        dimension=0,
        num_keys=1,
        is_stable=True,
    )
    ssc = -key
    npadded = _NPAD - _N
    px = jnp.full((npadded,), 1e7, jnp.float32)
    sx1 = jnp.concatenate([sx1, px])
    sy1 = jnp.concatenate([sy1, px])
    sx2 = jnp.concatenate([sx2, px + 1.0])
    sy2 = jnp.concatenate([sy2, px + 1.0])
    ssc = jnp.concatenate([ssc, jnp.zeros((npadded,), jnp.float32)])
    ordf = jnp.concatenate([ordf, jnp.arange(_N, _NPAD, dtype=jnp.float32)])
    bt = jnp.stack([sx1, sy1, sx2, sy2, ordf])
    bc = jnp.stack([sx1, sy1, sx2, sy2, ordf, ssc], axis=1)

    out8 = pl.pallas_call(
        _nms_body,
        out_shape=jax.ShapeDtypeStruct((_NPAD, 8), jnp.float32),
        scratch_shapes=[pltpu.VMEM((8, _NPAD), jnp.float32)],
    )(bt, bc)

    return out8[:_N, 0:5]


# speedup vs baseline: 2.0371x; 2.0371x over previous
import jax
import jax.numpy as jnp
from jax import lax
from jax.experimental import pallas as pl
from jax.experimental.pallas import tpu as pltpu

_N = 5000
_T = 512
_NBLK = 10
_NPAD = _T * _NBLK
_THR = 0.7


def _nms_body(bt_ref, bc_ref, bo_ref, out_ref, keep_ref):
    upper = (
        lax.broadcasted_iota(jnp.int32, (_T, _T), 0)
        < lax.broadcasted_iota(jnp.int32, (_T, _T), 1)
    ).astype(jnp.float32)
    colf = lax.broadcasted_iota(jnp.int32, (1, _NPAD), 1).astype(jnp.float32)
    keep_ref[...] = jnp.ones((8, _NPAD), jnp.float32)
    keep_ref[1:2, :] = jnp.zeros((1, _NPAD), jnp.float32)

    _C = _THR / (1.0 + _THR)
    x1f = bt_ref[0:1, :]
    y1f = bt_ref[1:2, :]
    x2f = bt_ref[2:3, :]
    y2f = bt_ref[3:4, :]
    careaf = _C * ((x2f - x1f) * (y2f - y1f))

    for j in range(_NBLK):
        b = j * _T
        x1 = x1f[:, b:]
        y1 = y1f[:, b:]
        x2 = x2f[:, b:]
        y2 = y2f[:, b:]
        carea = careaf[:, b:]
        rx1 = bc_ref[b : b + _T, 0:1]
        ry1 = bc_ref[b : b + _T, 1:2]
        rx2 = bc_ref[b : b + _T, 2:3]
        ry2 = bc_ref[b : b + _T, 3:4]
        crarea = _C * ((rx2 - rx1) * (ry2 - ry1))
        xl = jnp.minimum(rx2, x2) - jnp.maximum(rx1, x1)
        yl = jnp.minimum(ry2, y2) - jnp.maximum(ry1, y1)
        inter = xl * jnp.maximum(yl, 0.0)
        smat = (inter > crarea + carea).astype(jnp.float32)

        diag = smat[:, 0:_T] * upper
        kb0 = keep_ref[0:1, b : b + _T]

        def cond(c):
            return c[2]

        def body(c):
            kb, _, _ = c
            s = lax.dot(kb, diag, preferred_element_type=jnp.float32)
            kbn = jnp.where(s > 0.0, 0.0, kb0)
            return (kbn, kb, jnp.any(kbn != kb))

        kb = lax.while_loop(cond, body, (kb0, kb0, jnp.bool_(True)))[0]

        keep_ref[0:1, b : b + _T] = kb
        if j < _NBLK - 1:
            sup = lax.dot(kb, smat, preferred_element_type=jnp.float32)
            lcol = lax.broadcasted_iota(jnp.int32, (1, _NPAD - b), 1)
            keep = keep_ref[0:1, b:]
            keep_ref[0:1, b:] = jnp.where(
                (lcol >= _T) & (sup > 0.0), 0.0, keep
            )

        ocf = bc_ref[b : b + _T, 4:5]
        ptile = (ocf == colf).astype(jnp.float32)
        keep_ref[1:2, :] += lax.dot(kb, ptile, preferred_element_type=jnp.float32)

    kc = jnp.reshape(keep_ref[1:2, :], (_NPAD, 1))
    out_ref[...] = bo_ref[...] * kc


def kernel(boxes, scores):
    iota = jnp.arange(_N, dtype=jnp.float32)
    _, ordf, sx1, sy1, sx2, sy2 = lax.sort(
        (-scores, iota, boxes[:, 0], boxes[:, 1], boxes[:, 2], boxes[:, 3]),
        dimension=0,
        num_keys=1,
        is_stable=True,
    )
    npadded = _NPAD - _N
    px = jnp.full((npadded,), 1e7, jnp.float32)
    sx1 = jnp.concatenate([sx1, px])
    sy1 = jnp.concatenate([sy1, px])
    sx2 = jnp.concatenate([sx2, px + 1.0])
    sy2 = jnp.concatenate([sy2, px + 1.0])
    ordf = jnp.concatenate([ordf, jnp.arange(_N, _NPAD, dtype=jnp.float32)])
    bt = jnp.stack([sx1, sy1, sx2, sy2])
    bc = jnp.stack([sx1, sy1, sx2, sy2, ordf], axis=1)
    bo = jnp.concatenate(
        [boxes, scores[:, None]], axis=1
    )
    bo = jnp.concatenate([bo, jnp.zeros((npadded, 5), jnp.float32)], axis=0)

    out8 = pl.pallas_call(
        _nms_body,
        out_shape=jax.ShapeDtypeStruct((_NPAD, 5), jnp.float32),
        scratch_shapes=[pltpu.VMEM((8, _NPAD), jnp.float32)],
    )(bt, bc, bo)

    return out8[:_N, :]


# speedup vs baseline: 2.2420x; 1.1006x over previous
import jax
import jax.numpy as jnp
from jax import lax
from jax.experimental import pallas as pl

_N = 5000
_T = 512
_NBLK = 10
_NPAD = _T * _NBLK
_THR = 0.7


def _nms_body(bt_ref, bc_ref, keep_ref):
    upper = (
        lax.broadcasted_iota(jnp.int32, (_T, _T), 0)
        < lax.broadcasted_iota(jnp.int32, (_T, _T), 1)
    ).astype(jnp.float32)
    colf = lax.broadcasted_iota(jnp.int32, (1, _NPAD), 1).astype(jnp.float32)
    keep_ref[...] = jnp.zeros((8, _NPAD), jnp.float32)
    keep_ref[0:1, :] = jnp.ones((1, _NPAD), jnp.float32)

    _C = _THR / (1.0 + _THR)
    x1f = bt_ref[0:1, :]
    y1f = bt_ref[1:2, :]
    x2f = bt_ref[2:3, :]
    y2f = bt_ref[3:4, :]
    careaf = _C * ((x2f - x1f) * (y2f - y1f))

    for j in range(_NBLK):
        b = j * _T
        x1 = x1f[:, b:]
        y1 = y1f[:, b:]
        x2 = x2f[:, b:]
        y2 = y2f[:, b:]
        carea = careaf[:, b:]
        rx1 = bc_ref[b : b + _T, 0:1]
        ry1 = bc_ref[b : b + _T, 1:2]
        rx2 = bc_ref[b : b + _T, 2:3]
        ry2 = bc_ref[b : b + _T, 3:4]
        crarea = _C * ((rx2 - rx1) * (ry2 - ry1))
        xl = jnp.minimum(rx2, x2) - jnp.maximum(rx1, x1)
        yl = jnp.minimum(ry2, y2) - jnp.maximum(ry1, y1)
        inter = xl * jnp.maximum(yl, 0.0)
        smat = (inter > crarea + carea).astype(jnp.float32)

        diag = smat[:, 0:_T] * upper
        kb0 = keep_ref[0:1, b : b + _T]

        def cond(c):
            return c[2]

        def body(c):
            kb, _, _ = c
            s = lax.dot(kb, diag, preferred_element_type=jnp.float32)
            kbn = jnp.where(s > 0.0, 0.0, kb0)
            return (kbn, kb, jnp.any(kbn != kb))

        kb = lax.while_loop(cond, body, (kb0, kb0, jnp.bool_(True)))[0]

        keep_ref[0:1, b : b + _T] = kb
        ocf = bc_ref[b : b + _T, 4:5]
        ptile = (ocf == colf).astype(jnp.float32)
        keep_ref[1:2, :] += lax.dot(kb, ptile, preferred_element_type=jnp.float32)
        if j < _NBLK - 1:
            sup = lax.dot(kb, smat, preferred_element_type=jnp.float32)
            lcol = lax.broadcasted_iota(jnp.int32, (1, _NPAD - b), 1)
            keep = keep_ref[0:1, b:]
            keep_ref[0:1, b:] = jnp.where(
                (lcol >= _T) & (sup > 0.0), 0.0, keep
            )


def kernel(boxes, scores):
    iota = jnp.arange(_N, dtype=jnp.float32)
    _, ordf, sx1, sy1, sx2, sy2 = lax.sort(
        (-scores, iota, boxes[:, 0], boxes[:, 1], boxes[:, 2], boxes[:, 3]),
        dimension=0,
        num_keys=1,
        is_stable=True,
    )
    npadded = _NPAD - _N
    px = jnp.full((npadded,), 1e7, jnp.float32)
    sx1 = jnp.concatenate([sx1, px])
    sy1 = jnp.concatenate([sy1, px])
    sx2 = jnp.concatenate([sx2, px + 1.0])
    sy2 = jnp.concatenate([sy2, px + 1.0])
    ordf = jnp.concatenate([ordf, jnp.arange(_N, _NPAD, dtype=jnp.float32)])
    bt = jnp.stack([sx1, sy1, sx2, sy2])
    bc = jnp.stack([sx1, sy1, sx2, sy2, ordf], axis=1)

    keep8 = pl.pallas_call(
        _nms_body,
        out_shape=jax.ShapeDtypeStruct((8, _NPAD), jnp.float32),
    )(bt, bc)

    keep = keep8[1, :_N]
    out_boxes = boxes * keep[:, None]
    out_scores = scores * keep
    return jnp.concatenate([out_boxes, out_scores[:, None]], axis=1)
